# baseline (device time: 44955 ns/iter reference)
import jax
import jax.numpy as jnp
from jax import lax
from jax.experimental import pallas as pl
from jax.experimental.pallas import tpu as pltpu

N_DEV = 16
HQ = 8
DH = 128
SQ = 256
D = HQ * DH
SCALE = 0.08838834764831843 * 1.4426950408889634
CW = D + 128

RS_HALF = (128, 64, 32, 16)
RS_OFF = (0, 128, 192, 224)
STAGE_ROWS = 240
CHUNK = SQ // N_DEV


def kernel(x, Wq, Wo, K_ext, V_ext):
    skv = K_ext.shape[1]

    def body(x_ref, wq_ref, wo_ref, k_hbm, v_hbm, out_ref,
             acc_ref, sstage_ref, rstage_ref, gbuf_ref, q_ref,
             kb_ref, vb_ref, kstage_ref, vstage_ref,
             send_sems, recv_sems, kv_sems, ag_send_sems, ag_recv_sems):
        my = lax.axis_index("i")

        barrier = pltpu.get_barrier_semaphore()
        for m in range(1, N_DEV):
            pl.semaphore_signal(barrier, inc=1, device_id=(my ^ m,),
                                device_id_type=pl.DeviceIdType.MESH)

        def kv_dma(h, slot):
            ck = pltpu.make_async_copy(
                k_hbm.at[0, :, h, :], kstage_ref.at[slot],
                kv_sems.at[2 * slot])
            cv = pltpu.make_async_copy(
                v_hbm.at[0, :, h, :], vstage_ref.at[slot],
                kv_sems.at[2 * slot + 1])
            ck.start()
            cv.start()
            return ck, cv

        pends = {h: kv_dma(h, h) for h in range(HQ)}

        q = lax.dot_general(x_ref[0].astype(jnp.bfloat16),
                            wq_ref[...].astype(jnp.bfloat16),
                            (((1,), (0,)), ((), ())),
                            preferred_element_type=jnp.float32)
        q_ref[...] = (q * SCALE).astype(jnp.bfloat16)

        b0 = my & 1
        keep_lo = b0 * 128
        send_lo = (1 - b0) * 128

        qd = q_ref[pl.ds(send_lo, 128), :]
        ls = []
        for h in range(HQ):
            ck, cv = pends.pop(h)
            ck.wait()
            cv.wait()
            slot = h
            kb = kstage_ref[slot].astype(jnp.bfloat16)
            vb = vstage_ref[slot].astype(jnp.bfloat16)
            kb_ref[:, h * DH:(h + 1) * DH] = kb
            vb_ref[:, h * DH:(h + 1) * DH] = vb
            s = lax.dot_general(qd[:, h * DH:(h + 1) * DH], kb,
                                (((1,), (1,)), ((), ())),
                                preferred_element_type=jnp.float32)
            p = jnp.exp2(s)
            ls.append(jnp.sum(p, axis=1, keepdims=True))
            acc_h = lax.dot_general(p.astype(jnp.bfloat16), vb,
                                    (((1,), (0,)), ((), ())),
                                    preferred_element_type=jnp.float32)
            acc_ref[pl.ds(send_lo, 128), h * DH:(h + 1) * DH] = acc_h
        lblock = jnp.concatenate(
            ls + [jnp.zeros((128, 128 - HQ), jnp.float32)], axis=1)
        acc_ref[pl.ds(send_lo, 128), D:] = lblock

        def attn_block(lo, nrows):
            qrows = q_ref[pl.ds(lo, nrows), :]
            ls = []
            for h in range(HQ):
                s = lax.dot_general(qrows[:, h * DH:(h + 1) * DH],
                                    kb_ref[:, h * DH:(h + 1) * DH],
                                    (((1,), (1,)), ((), ())),
                                    preferred_element_type=jnp.float32)
                p = jnp.exp2(s)
                ls.append(jnp.sum(p, axis=1, keepdims=True))
                acc_h = lax.dot_general(p.astype(jnp.bfloat16),
                                        vb_ref[:, h * DH:(h + 1) * DH],
                                        (((1,), (0,)), ((), ())),
                                        preferred_element_type=jnp.float32)
                acc_ref[pl.ds(lo, nrows), h * DH:(h + 1) * DH] = acc_h
            lblock = jnp.concatenate(
                ls + [jnp.zeros((nrows, 128 - HQ), jnp.float32)], axis=1)
            acc_ref[pl.ds(lo, nrows), D:] = lblock

        def exchange(step, src_slice, dst_slice, partner):
            return pltpu.make_async_remote_copy(
                src_ref=src_slice,
                dst_ref=dst_slice,
                send_sem=send_sems.at[step],
                recv_sem=recv_sems.at[step],
                device_id=(partner,),
                device_id_type=pl.DeviceIdType.MESH,
            )

        def rs_start(k, slo):
            half = RS_HALF[k]
            sstage_ref[pl.ds(RS_OFF[k], half), :] = (
                acc_ref[pl.ds(slo, half), :].astype(jnp.bfloat16))
            rdma = exchange(k, sstage_ref.at[pl.ds(RS_OFF[k], half)],
                            rstage_ref.at[pl.ds(RS_OFF[k], half)],
                            my ^ (1 << k))
            rdma.start()
            return rdma

        def rs_finish(k, rdma, klo):
            half = RS_HALF[k]
            rdma.wait()
            acc_ref[pl.ds(klo, half), :] = (
                acc_ref[pl.ds(klo, half), :]
                + rstage_ref[pl.ds(RS_OFF[k], half), :].astype(jnp.float32))

        pl.semaphore_wait(barrier, N_DEV - 1)
        rdma0 = rs_start(0, send_lo)
        attn_block(keep_lo, 128)
        rs_finish(0, rdma0, keep_lo)

        mylo = keep_lo + (my >> 1) * CHUNK
        rdmas = []
        for m in (2, 4, 6, 8, 10, 12, 14):
            idx = m // 2 - 1
            partner = my ^ m
            plo = keep_lo + (partner >> 1) * CHUNK
            off = 128 + idx * CHUNK
            sstage_ref[pl.ds(off, CHUNK), :] = (
                acc_ref[pl.ds(plo, CHUNK), :].astype(jnp.bfloat16))
            r = exchange(1 + idx, sstage_ref.at[pl.ds(off, CHUNK)],
                         rstage_ref.at[pl.ds(off, CHUNK)], partner)
            r.start()
            rdmas.append((r, off))
        red = acc_ref[pl.ds(mylo, CHUNK), :]
        for r, off in rdmas:
            r.wait()
            red = red + rstage_ref[pl.ds(off, CHUNK), :].astype(jnp.float32)

        linv = 1.0 / red[:, D:D + HQ]
        outs = []
        for h in range(HQ):
            o = red[:, h * DH:(h + 1) * DH] * linv[:, h:h + 1]
            outs.append(o.astype(jnp.bfloat16))
        o16 = jnp.concatenate(outs, axis=1)
        p16 = lax.dot_general(o16, wo_ref[...].astype(jnp.bfloat16),
                              (((1,), (0,)), ((), ())),
                              preferred_element_type=jnp.float32)
        gbuf_ref[pl.ds(mylo, CHUNK), :] = p16.astype(jnp.bfloat16)

        ag = []
        for m in range(1, N_DEV):
            r = pltpu.make_async_remote_copy(
                src_ref=gbuf_ref.at[pl.ds(mylo, CHUNK)],
                dst_ref=gbuf_ref.at[pl.ds(mylo, CHUNK)],
                send_sem=ag_send_sems.at[m - 1],
                recv_sem=ag_recv_sems.at[m - 1],
                device_id=(my ^ m,),
                device_id_type=pl.DeviceIdType.MESH,
            )
            r.start()
            ag.append(r)
        for r in ag:
            r.wait()

        out_ref[0] = gbuf_ref[...].astype(jnp.float32)

    return pl.pallas_call(
        body,
        out_shape=jax.ShapeDtypeStruct((1, SQ, D), jnp.float32),
        in_specs=[
            pl.BlockSpec(memory_space=pltpu.VMEM),
            pl.BlockSpec(memory_space=pltpu.VMEM),
            pl.BlockSpec(memory_space=pltpu.VMEM),
            pl.BlockSpec(memory_space=pltpu.MemorySpace.HBM),
            pl.BlockSpec(memory_space=pltpu.MemorySpace.HBM),
        ],
        out_specs=pl.BlockSpec(memory_space=pltpu.VMEM),
        scratch_shapes=[
            pltpu.VMEM((SQ, CW), jnp.float32),
            pltpu.VMEM((STAGE_ROWS, CW), jnp.bfloat16),
            pltpu.VMEM((STAGE_ROWS, CW), jnp.bfloat16),
            pltpu.VMEM((SQ, D), jnp.bfloat16),
            pltpu.VMEM((SQ, D), jnp.bfloat16),
            pltpu.VMEM((skv, D), jnp.bfloat16),
            pltpu.VMEM((skv, D), jnp.bfloat16),
            pltpu.VMEM((HQ, skv, DH), jnp.float32),
            pltpu.VMEM((HQ, skv, DH), jnp.float32),
            pltpu.SemaphoreType.DMA((8,)),
            pltpu.SemaphoreType.DMA((8,)),
            pltpu.SemaphoreType.DMA((2 * HQ,)),
            pltpu.SemaphoreType.DMA((N_DEV - 1,)),
            pltpu.SemaphoreType.DMA((N_DEV - 1,)),
        ],
        compiler_params=pltpu.CompilerParams(
            collective_id=0, vmem_limit_bytes=100 * 1024 * 1024),
    )(x, Wq, Wo, K_ext, V_ext)


# device time: 44266 ns/iter; 1.0156x vs baseline; 1.0156x over previous
import jax
import jax.numpy as jnp
from jax import lax
from jax.experimental import pallas as pl
from jax.experimental.pallas import tpu as pltpu

N_DEV = 16
HQ = 8
DH = 128
SQ = 256
D = HQ * DH
SCALE = 0.08838834764831843 * 1.4426950408889634
CW = D + 128

RS_HALF = (128, 64, 32, 16)
RS_OFF = (0, 128, 192, 224)
STAGE_ROWS = 240
CHUNK = SQ // N_DEV


def kernel(x, Wq, Wo, K_ext, V_ext):
    skv = K_ext.shape[1]

    def body(x_ref, wq_ref, wo_ref, k_hbm, v_hbm, out_ref,
             acc_ref, sstage_ref, rstage_ref, gbuf_ref, q_ref,
             kb_ref, vb_ref, kstage_ref, vstage_ref,
             send_sems, recv_sems, kv_sems, ag_send_sems, ag_recv_sems):
        my = lax.axis_index("i")

        barrier = pltpu.get_barrier_semaphore()
        for m in range(1, N_DEV):
            pl.semaphore_signal(barrier, inc=1, device_id=(my ^ m,),
                                device_id_type=pl.DeviceIdType.MESH)

        def kv_dma(h, slot):
            ck = pltpu.make_async_copy(
                k_hbm.at[0, :, h, :], kstage_ref.at[slot],
                kv_sems.at[2 * slot])
            cv = pltpu.make_async_copy(
                v_hbm.at[0, :, h, :], vstage_ref.at[slot],
                kv_sems.at[2 * slot + 1])
            ck.start()
            cv.start()
            return ck, cv

        pends = {0: kv_dma(0, 0), 1: kv_dma(1, 1)}

        q = lax.dot_general(x_ref[0].astype(jnp.bfloat16),
                            wq_ref[...].astype(jnp.bfloat16),
                            (((1,), (0,)), ((), ())),
                            preferred_element_type=jnp.float32)
        q_ref[...] = (q * SCALE).astype(jnp.bfloat16)

        b0 = my & 1
        keep_lo = b0 * 128
        send_lo = (1 - b0) * 128

        qd = q_ref[pl.ds(send_lo, 128), :]
        ls = []
        for h in range(HQ):
            ck, cv = pends.pop(h)
            ck.wait()
            cv.wait()
            if h + 2 < HQ:
                pends[h + 2] = kv_dma(h + 2, (h + 2) % 4)
            slot = h % 4
            kb = kstage_ref[slot].astype(jnp.bfloat16)
            vb = vstage_ref[slot].astype(jnp.bfloat16)
            kb_ref[:, h * DH:(h + 1) * DH] = kb
            vb_ref[:, h * DH:(h + 1) * DH] = vb
            s = lax.dot_general(qd[:, h * DH:(h + 1) * DH], kb,
                                (((1,), (1,)), ((), ())),
                                preferred_element_type=jnp.float32)
            p = jnp.exp2(s)
            ls.append(jnp.sum(p, axis=1, keepdims=True))
            acc_h = lax.dot_general(p.astype(jnp.bfloat16), vb,
                                    (((1,), (0,)), ((), ())),
                                    preferred_element_type=jnp.float32)
            acc_ref[pl.ds(send_lo, 128), h * DH:(h + 1) * DH] = acc_h
        lblock = jnp.concatenate(
            ls + [jnp.zeros((128, 128 - HQ), jnp.float32)], axis=1)
        acc_ref[pl.ds(send_lo, 128), D:] = lblock

        def attn_block(lo, nrows):
            qrows = q_ref[pl.ds(lo, nrows), :]
            ls = []
            for h in range(HQ):
                s = lax.dot_general(qrows[:, h * DH:(h + 1) * DH],
                                    kb_ref[:, h * DH:(h + 1) * DH],
                                    (((1,), (1,)), ((), ())),
                                    preferred_element_type=jnp.float32)
                p = jnp.exp2(s)
                ls.append(jnp.sum(p, axis=1, keepdims=True))
                acc_h = lax.dot_general(p.astype(jnp.bfloat16),
                                        vb_ref[:, h * DH:(h + 1) * DH],
                                        (((1,), (0,)), ((), ())),
                                        preferred_element_type=jnp.float32)
                acc_ref[pl.ds(lo, nrows), h * DH:(h + 1) * DH] = acc_h
            lblock = jnp.concatenate(
                ls + [jnp.zeros((nrows, 128 - HQ), jnp.float32)], axis=1)
            acc_ref[pl.ds(lo, nrows), D:] = lblock

        def exchange(step, src_slice, dst_slice, partner):
            return pltpu.make_async_remote_copy(
                src_ref=src_slice,
                dst_ref=dst_slice,
                send_sem=send_sems.at[step],
                recv_sem=recv_sems.at[step],
                device_id=(partner,),
                device_id_type=pl.DeviceIdType.MESH,
            )

        def rs_start(k, slo):
            half = RS_HALF[k]
            sstage_ref[pl.ds(RS_OFF[k], half), :] = (
                acc_ref[pl.ds(slo, half), :].astype(jnp.bfloat16))
            rdma = exchange(k, sstage_ref.at[pl.ds(RS_OFF[k], half)],
                            rstage_ref.at[pl.ds(RS_OFF[k], half)],
                            my ^ (1 << k))
            rdma.start()
            return rdma

        def rs_finish(k, rdma, klo):
            half = RS_HALF[k]
            rdma.wait()
            acc_ref[pl.ds(klo, half), :] = (
                acc_ref[pl.ds(klo, half), :]
                + rstage_ref[pl.ds(RS_OFF[k], half), :].astype(jnp.float32))

        pl.semaphore_wait(barrier, N_DEV - 1)
        rdma0 = rs_start(0, send_lo)
        attn_block(keep_lo, 128)
        rs_finish(0, rdma0, keep_lo)

        mylo = keep_lo + (my >> 1) * CHUNK
        rdmas = []
        for m in (2, 4, 6, 8, 10, 12, 14):
            idx = m // 2 - 1
            partner = my ^ m
            plo = keep_lo + (partner >> 1) * CHUNK
            off = 128 + idx * CHUNK
            sstage_ref[pl.ds(off, CHUNK), :] = (
                acc_ref[pl.ds(plo, CHUNK), :].astype(jnp.bfloat16))
            r = exchange(1 + idx, sstage_ref.at[pl.ds(off, CHUNK)],
                         rstage_ref.at[pl.ds(off, CHUNK)], partner)
            r.start()
            rdmas.append((r, off))
        red = acc_ref[pl.ds(mylo, CHUNK), :]
        for r, off in rdmas:
            r.wait()
            red = red + rstage_ref[pl.ds(off, CHUNK), :].astype(jnp.float32)

        linv = 1.0 / red[:, D:D + HQ]
        outs = []
        for h in range(HQ):
            o = red[:, h * DH:(h + 1) * DH] * linv[:, h:h + 1]
            outs.append(o.astype(jnp.bfloat16))
        o16 = jnp.concatenate(outs, axis=1)
        p16 = lax.dot_general(o16, wo_ref[...].astype(jnp.bfloat16),
                              (((1,), (0,)), ((), ())),
                              preferred_element_type=jnp.float32)
        gbuf_ref[pl.ds(mylo, CHUNK), :] = p16.astype(jnp.bfloat16)

        ag = []
        for m in range(1, N_DEV):
            r = pltpu.make_async_remote_copy(
                src_ref=gbuf_ref.at[pl.ds(mylo, CHUNK)],
                dst_ref=gbuf_ref.at[pl.ds(mylo, CHUNK)],
                send_sem=ag_send_sems.at[m - 1],
                recv_sem=ag_recv_sems.at[m - 1],
                device_id=(my ^ m,),
                device_id_type=pl.DeviceIdType.MESH,
            )
            r.start()
            ag.append(r)
        for r in ag:
            r.wait()

        out_ref[0] = gbuf_ref[...].astype(jnp.float32)

    return pl.pallas_call(
        body,
        out_shape=jax.ShapeDtypeStruct((1, SQ, D), jnp.float32),
        in_specs=[
            pl.BlockSpec(memory_space=pltpu.VMEM),
            pl.BlockSpec(memory_space=pltpu.VMEM),
            pl.BlockSpec(memory_space=pltpu.VMEM),
            pl.BlockSpec(memory_space=pltpu.MemorySpace.HBM),
            pl.BlockSpec(memory_space=pltpu.MemorySpace.HBM),
        ],
        out_specs=pl.BlockSpec(memory_space=pltpu.VMEM),
        scratch_shapes=[
            pltpu.VMEM((SQ, CW), jnp.float32),
            pltpu.VMEM((STAGE_ROWS, CW), jnp.bfloat16),
            pltpu.VMEM((STAGE_ROWS, CW), jnp.bfloat16),
            pltpu.VMEM((SQ, D), jnp.bfloat16),
            pltpu.VMEM((SQ, D), jnp.bfloat16),
            pltpu.VMEM((skv, D), jnp.bfloat16),
            pltpu.VMEM((skv, D), jnp.bfloat16),
            pltpu.VMEM((4, skv, DH), jnp.float32),
            pltpu.VMEM((4, skv, DH), jnp.float32),
            pltpu.SemaphoreType.DMA((8,)),
            pltpu.SemaphoreType.DMA((8,)),
            pltpu.SemaphoreType.DMA((8,)),
            pltpu.SemaphoreType.DMA((N_DEV - 1,)),
            pltpu.SemaphoreType.DMA((N_DEV - 1,)),
        ],
        compiler_params=pltpu.CompilerParams(
            collective_id=0, vmem_limit_bytes=100 * 1024 * 1024),
    )(x, Wq, Wo, K_ext, V_ext)
